# baseline (speedup 1.0000x reference)
import jax
import jax.numpy as jnp
from jax import lax
from jax.experimental import pallas as pl
from jax.experimental.pallas import tpu as pltpu
from jax.experimental.pallas import tpu_sc as plsc

F32 = jnp.float32

_GDN = None


def _lane_perm(x, perm):
    return lax.gather(
        x, perm.reshape(16, 1),
        lax.GatherDimensionNumbers(offset_dims=(),
                                   collapsed_slice_dims=(0,),
                                   start_index_map=(0,)),
        (1,), mode=lax.GatherScatterMode.PROMISE_IN_BOUNDS)


def _allsum_lanes(v):
    for s in (8, 4, 2, 1):
        perm = jnp.arange(16, dtype=jnp.int32) ^ s
        v = v + _lane_perm(v, perm)
    return v

B = 16384
K = 16
F = 128
NW = 32
EPB = B // NW
CH = 64
NCHUNK = EPB // CH
NBUF = 2


def _sc_agg_body(table, nodes, nghtw, self_out, agg_out, env_out,
                 nidf_v, nght_v, acc_v, self_v, oself_v, oagg_v, oenv_v,
                 inv_v, cdiv_v, sem_g, sem_o):
    wid = lax.axis_index("s") * 2 + lax.axis_index("c")
    base = wid * EPB

    pltpu.sync_copy(nodes.at[pl.ds(base, EPB)], nidf_v)
    pltpu.sync_copy(nghtw.at[wid], nght_v)

    def cnt_body(g, _):
        sl = pl.ds(g * 16, 16)
        nidv = nidf_v[sl]
        cacc = jnp.zeros((16,), jnp.int32)
        for k in range(K):
            cacc = cacc + jnp.where(nght_v[k, sl] == nidv, 1, 0)
        env_cnt = jnp.maximum(K - cacc, 1).astype(F32)
        ie = 1.0 / env_cnt
        inv_v[sl] = ie
        cdiv_v[sl] = cacc.astype(F32) * ie
        return 0

    lax.fori_loop(0, EPB // 16, cnt_body, 0)

    def zero_body(e, _):
        z = jnp.zeros((16,), F32)
        for b in range(NBUF):
            for j in range(F // 16):
                acc_v[b, e, pl.ds(j * 16, 16)] = z
        return 0

    lax.fori_loop(0, CH, zero_body, 0)

    def in_descs(gi, b):
        ds = pl.ds(gi * CH, CH)
        descs = [pltpu.make_async_copy(
            table.at[nght_v.at[k, ds]], acc_v.at[b], sem_g)
            for k in range(K)]
        descs.append(pltpu.make_async_copy(
            table.at[nidf_v.at[ds]], self_v.at[b], sem_g))
        return descs

    def fire_in(gi, b):
        descs = in_descs(gi, b)
        for d in descs[:K]:
            d.start(add=True)
        descs[K].start()

    def out_descs(gi, b):
        dst = pl.ds(base + gi * CH, CH)
        return (
            pltpu.make_async_copy(oself_v.at[b], self_out.at[dst], sem_o),
            pltpu.make_async_copy(oagg_v.at[b], agg_out.at[dst], sem_o),
            pltpu.make_async_copy(oenv_v.at[b], env_out.at[dst], sem_o),
        )

    def compute(gi, b, ob):
        def ebody(e, _):
            g16 = gi * (CH // 16) + e // 16
            lane = e % 16
            ie = _lane_perm(inv_v[pl.ds(g16 * 16, 16)],
                            jnp.full((16,), lane, jnp.int32))
            ce = _lane_perm(cdiv_v[pl.ds(g16 * 16, 16)],
                            jnp.full((16,), lane, jnp.int32))
            z = jnp.zeros((16,), F32)
            for j in range(F // 16):
                sl = pl.ds(j * 16, 16)
                s = acc_v[b, e, sl]
                selfj = self_v[b, e, sl]
                oself_v[ob, e, sl] = selfj
                oagg_v[ob, e, sl] = s * (1.0 / K)
                oenv_v[ob, e, sl] = s * ie - selfj * ce
                acc_v[b, e, sl] = z
            return 0

        lax.fori_loop(0, CH, ebody, 0)

    for g in range(NBUF - 1):
        fire_in(g, g)

    def ring_body(p, _):
        for bb in range(NBUF):
            gi = p * NBUF + bb
            ob = bb % 2

            @pl.when(gi >= 2)
            def _():
                for d in out_descs(gi - 2, ob):
                    d.wait()

            @pl.when(gi + NBUF - 1 < NCHUNK)
            def _():
                fire_in(gi + NBUF - 1, (bb + NBUF - 1) % NBUF)

            for d in in_descs(gi, bb):
                d.wait()
            compute(gi, bb, ob)
            for d in out_descs(gi, ob):
                d.start()
        return 0

    lax.fori_loop(0, NCHUNK // NBUF, ring_body, 0)
    for gi, ob in ((NCHUNK - 2, 0), (NCHUNK - 1, 1)):
        for d in out_descs(gi, ob):
            d.wait()


def _sc_agg(table, nodes, nghtw):
    mesh = plsc.VectorSubcoreMesh(core_axis_name="c", subcore_axis_name="s")
    out_t = [jax.ShapeDtypeStruct((B, F), F32)] * 3
    scratch = [
        pltpu.VMEM((EPB,), jnp.int32),
        pltpu.VMEM((K, EPB), jnp.int32),
        pltpu.VMEM((NBUF, CH, F), F32),
        pltpu.VMEM((NBUF, CH, F), F32),
        pltpu.VMEM((2, CH, F), F32),
        pltpu.VMEM((2, CH, F), F32),
        pltpu.VMEM((2, CH, F), F32),
        pltpu.VMEM((EPB,), F32),
        pltpu.VMEM((EPB,), F32),
        pltpu.SemaphoreType.DMA,
        pltpu.SemaphoreType.DMA,
    ]
    fn = pl.kernel(_sc_agg_body, out_type=out_t, mesh=mesh,
                   scratch_types=scratch)
    return fn(table, nodes, nghtw)


BM = 4096


def _tc1_body(self_ref, agg_ref, env_ref, wg1_ref, bg1_ref, wg2_ref, bg2_ref,
              raw_ref, gen_ref, eraw_ref, egen_ref, ssum_ref, ssq_ref):
    cat = jnp.concatenate([self_ref[...], agg_ref[...]], axis=1)
    h = jnp.maximum(
        jnp.dot(cat, wg1_ref[...], preferred_element_type=F32) + bg1_ref[...],
        0.0)
    raw_ref[...] = h
    g = jnp.dot(h, wg2_ref[...], preferred_element_type=F32) + bg2_ref[...]
    gen_ref[...] = g
    eh = jnp.maximum(
        jnp.dot(env_ref[...], wg1_ref[F:, :], preferred_element_type=F32)
        + bg1_ref[...], 0.0)
    eraw_ref[...] = eh
    egen_ref[...] = (jnp.dot(eh, wg2_ref[...], preferred_element_type=F32)
                     + bg2_ref[...])

    a = agg_ref[...]
    s1 = jnp.concatenate([jnp.sum(a, 0), jnp.sum(g, 0)]).reshape(1, 3 * F)
    s2 = jnp.concatenate([jnp.sum(a * a, 0),
                          jnp.sum(g * g, 0)]).reshape(1, 3 * F)

    @pl.when(pl.program_id(0) == 0)
    def _():
        ssum_ref[...] = jnp.zeros_like(ssum_ref)
        ssq_ref[...] = jnp.zeros_like(ssq_ref)

    ssum_ref[...] += s1
    ssq_ref[...] += s2


def _tc1(self_f, agg, env, wg1, bg1, wg2, bg2):
    grid = (B // BM,)
    return pl.pallas_call(
        _tc1_body,
        grid=grid,
        in_specs=[
            pl.BlockSpec((BM, F), lambda i: (i, 0)),
            pl.BlockSpec((BM, F), lambda i: (i, 0)),
            pl.BlockSpec((BM, F), lambda i: (i, 0)),
            pl.BlockSpec((2 * F, 2 * F), lambda i: (0, 0)),
            pl.BlockSpec((1, 2 * F), lambda i: (0, 0)),
            pl.BlockSpec((2 * F, 2 * F), lambda i: (0, 0)),
            pl.BlockSpec((1, 2 * F), lambda i: (0, 0)),
        ],
        out_specs=[
            pl.BlockSpec((BM, 2 * F), lambda i: (i, 0)),
            pl.BlockSpec((BM, 2 * F), lambda i: (i, 0)),
            pl.BlockSpec((BM, 2 * F), lambda i: (i, 0)),
            pl.BlockSpec((BM, 2 * F), lambda i: (i, 0)),
            pl.BlockSpec((1, 3 * F), lambda i: (0, 0)),
            pl.BlockSpec((1, 3 * F), lambda i: (0, 0)),
        ],
        out_shape=[
            jax.ShapeDtypeStruct((B, 2 * F), F32),
            jax.ShapeDtypeStruct((B, 2 * F), F32),
            jax.ShapeDtypeStruct((B, 2 * F), F32),
            jax.ShapeDtypeStruct((B, 2 * F), F32),
            jax.ShapeDtypeStruct((1, 3 * F), F32),
            jax.ShapeDtypeStruct((1, 3 * F), F32),
        ],
    )(self_f, agg, env, wg1, bg1.reshape(1, -1), wg2, bg2.reshape(1, -1))


def _tc2_body(agg_ref, gen_ref, ssum_ref, ssq_ref, gamma_ref, beta_ref,
              w1_ref, out_ref):
    mu = ssum_ref[...] * (1.0 / B)
    var = ssq_ref[...] * (1.0 / B) - mu * mu
    scale = gamma_ref[...] / jnp.sqrt(var + 1e-5)
    x = jnp.concatenate([agg_ref[...], gen_ref[...]], axis=1)
    xn = (x - mu) * scale + beta_ref[...]
    out_ref[...] = jnp.maximum(
        jnp.dot(xn, w1_ref[...], preferred_element_type=F32), 0.0)


def _tc2(agg, gen, ssum, ssq, gamma, beta, w1):
    grid = (B // BM,)
    return pl.pallas_call(
        _tc2_body,
        grid=grid,
        in_specs=[
            pl.BlockSpec((BM, F), lambda i: (i, 0)),
            pl.BlockSpec((BM, 2 * F), lambda i: (i, 0)),
            pl.BlockSpec((1, 3 * F), lambda i: (0, 0)),
            pl.BlockSpec((1, 3 * F), lambda i: (0, 0)),
            pl.BlockSpec((1, 3 * F), lambda i: (0, 0)),
            pl.BlockSpec((1, 3 * F), lambda i: (0, 0)),
            pl.BlockSpec((3 * F, 3 * F), lambda i: (0, 0)),
        ],
        out_specs=pl.BlockSpec((BM, 3 * F), lambda i: (i, 0)),
        out_shape=jax.ShapeDtypeStruct((B, 3 * F), F32),
    )(agg, gen, ssum, ssq, gamma.reshape(1, -1), beta.reshape(1, -1), w1)


def kernel(table, nodes, to_neighs_list, Wg1, bg1, Wg2, bg2, W1, gamma1,
           beta1):
    assert table.shape == (100000, F) and nodes.shape == (B,)
    assert to_neighs_list.shape == (B, K)
    nghtw = (to_neighs_list.T.reshape(K, NW, EPB)
             .swapaxes(0, 1).reshape(NW, K, EPB))
    self_f, agg, env = _sc_agg(table, nodes, nghtw)
    raw, gen, eraw, egen, ssum, ssq = _tc1(self_f, agg, env, Wg1, bg1, Wg2,
                                           bg2)
    to_feats = _tc2(agg, gen, ssum, ssq, gamma1, beta1, W1)
    return (agg, to_feats, gen, raw, egen, eraw)


# speedup vs baseline: 1.0537x; 1.0537x over previous
import jax
import jax.numpy as jnp
from jax import lax
from jax.experimental import pallas as pl
from jax.experimental.pallas import tpu as pltpu
from jax.experimental.pallas import tpu_sc as plsc

F32 = jnp.float32

_GDN = None


def _lane_perm(x, perm):
    return lax.gather(
        x, perm.reshape(16, 1),
        lax.GatherDimensionNumbers(offset_dims=(),
                                   collapsed_slice_dims=(0,),
                                   start_index_map=(0,)),
        (1,), mode=lax.GatherScatterMode.PROMISE_IN_BOUNDS)


def _allsum_lanes(v):
    for s in (8, 4, 2, 1):
        perm = jnp.arange(16, dtype=jnp.int32) ^ s
        v = v + _lane_perm(v, perm)
    return v

B = 16384
K = 16
F = 128
NW = 32
EPB = B // NW
CH = 32
NCHUNK = EPB // CH
NBUF = 2


def _sc_agg_body(table, nodes, nghtw, self_out, agg_out, env_out,
                 nidf_v, nght_v, acc_v, self_v, oself_v, oagg_v, oenv_v,
                 inv_v, cdiv_v, sem_g, sem_o):
    wid = lax.axis_index("s") * 2 + lax.axis_index("c")
    base = wid * EPB

    pltpu.sync_copy(nodes.at[pl.ds(base, EPB)], nidf_v)
    pltpu.sync_copy(nghtw.at[wid], nght_v)

    def cnt_body(g, _):
        sl = pl.ds(g * 16, 16)
        nidv = nidf_v[sl]
        cacc = jnp.zeros((16,), jnp.int32)
        for k in range(K):
            cacc = cacc + jnp.where(nght_v[k, sl] == nidv, 1, 0)
        env_cnt = jnp.maximum(K - cacc, 1).astype(F32)
        ie = 1.0 / env_cnt
        inv_v[sl] = ie
        cdiv_v[sl] = cacc.astype(F32) * ie
        return 0

    lax.fori_loop(0, EPB // 16, cnt_body, 0)

    def zero_body(e, _):
        z = jnp.zeros((16,), F32)
        for b in range(NBUF):
            for j in range(F // 16):
                acc_v[b, e, pl.ds(j * 16, 16)] = z
        return 0

    lax.fori_loop(0, CH, zero_body, 0)

    def in_descs(gi, b):
        ds = pl.ds(gi * CH, CH)
        descs = [pltpu.make_async_copy(
            table.at[nght_v.at[k, ds]], acc_v.at[b], sem_g)
            for k in range(K)]
        descs.append(pltpu.make_async_copy(
            table.at[nidf_v.at[ds]], self_v.at[b], sem_g))
        return descs

    def fire_in(gi, b):
        descs = in_descs(gi, b)
        for d in descs[:K]:
            d.start(add=True)
        descs[K].start()

    def out_descs(gi, b):
        dst = pl.ds(base + gi * CH, CH)
        return (
            pltpu.make_async_copy(oself_v.at[b], self_out.at[dst], sem_o),
            pltpu.make_async_copy(oagg_v.at[b], agg_out.at[dst], sem_o),
            pltpu.make_async_copy(oenv_v.at[b], env_out.at[dst], sem_o),
        )

    def compute(gi, b, ob):
        def ebody(e, _):
            g16 = gi * (CH // 16) + e // 16
            lane = e % 16
            ie = _lane_perm(inv_v[pl.ds(g16 * 16, 16)],
                            jnp.full((16,), lane, jnp.int32))
            ce = _lane_perm(cdiv_v[pl.ds(g16 * 16, 16)],
                            jnp.full((16,), lane, jnp.int32))
            z = jnp.zeros((16,), F32)
            for j in range(F // 16):
                sl = pl.ds(j * 16, 16)
                s = acc_v[b, e, sl]
                selfj = self_v[b, e, sl]
                oself_v[ob, e, sl] = selfj
                oagg_v[ob, e, sl] = s * (1.0 / K)
                oenv_v[ob, e, sl] = s * ie - selfj * ce
                acc_v[b, e, sl] = z
            return 0

        lax.fori_loop(0, CH, ebody, 0)

    for g in range(NBUF - 1):
        fire_in(g, g)

    def ring_body(p, _):
        for bb in range(NBUF):
            gi = p * NBUF + bb
            ob = bb % 2

            @pl.when(gi >= 2)
            def _():
                for d in out_descs(gi - 2, ob):
                    d.wait()

            @pl.when(gi + NBUF - 1 < NCHUNK)
            def _():
                fire_in(gi + NBUF - 1, (bb + NBUF - 1) % NBUF)

            for d in in_descs(gi, bb):
                d.wait()
            compute(gi, bb, ob)
            for d in out_descs(gi, ob):
                d.start()
        return 0

    lax.fori_loop(0, NCHUNK // NBUF, ring_body, 0)
    for gi, ob in ((NCHUNK - 2, 0), (NCHUNK - 1, 1)):
        for d in out_descs(gi, ob):
            d.wait()


def _sc_agg(table, nodes, nghtw):
    mesh = plsc.VectorSubcoreMesh(core_axis_name="c", subcore_axis_name="s")
    out_t = [jax.ShapeDtypeStruct((B, F), F32)] * 3
    scratch = [
        pltpu.VMEM((EPB,), jnp.int32),
        pltpu.VMEM((K, EPB), jnp.int32),
        pltpu.VMEM((NBUF, CH, F), F32),
        pltpu.VMEM((NBUF, CH, F), F32),
        pltpu.VMEM((2, CH, F), F32),
        pltpu.VMEM((2, CH, F), F32),
        pltpu.VMEM((2, CH, F), F32),
        pltpu.VMEM((EPB,), F32),
        pltpu.VMEM((EPB,), F32),
        pltpu.SemaphoreType.DMA,
        pltpu.SemaphoreType.DMA,
    ]
    fn = pl.kernel(_sc_agg_body, out_type=out_t, mesh=mesh,
                   scratch_types=scratch)
    return fn(table, nodes, nghtw)


BM = 2048
NB2 = B // BM


def _tc_body(self_ref, agg_ref, env_ref, wg1_ref, bg1_ref, wg2_ref, bg2_ref,
             gamma_ref, beta_ref, w1_ref,
             raw_ref, gen_ref, eraw_ref, egen_ref, to_ref,
             gen_sc, ssum_ref, ssq_ref):
    p = pl.program_id(0)
    i = pl.program_id(1)

    @pl.when(p == 0)
    def _():
        cat = jnp.concatenate([self_ref[...], agg_ref[...]], axis=1)
        h = jnp.maximum(
            jnp.dot(cat, wg1_ref[...], preferred_element_type=F32)
            + bg1_ref[...], 0.0)
        raw_ref[...] = h
        g = (jnp.dot(h, wg2_ref[...], preferred_element_type=F32)
             + bg2_ref[...])
        gen_ref[...] = g
        gen_sc[pl.ds(i * BM, BM), :] = g
        eh = jnp.maximum(
            jnp.dot(env_ref[...], wg1_ref[F:, :], preferred_element_type=F32)
            + bg1_ref[...], 0.0)
        eraw_ref[...] = eh
        egen_ref[...] = (jnp.dot(eh, wg2_ref[...],
                                 preferred_element_type=F32) + bg2_ref[...])

        a = agg_ref[...]
        s1 = jnp.concatenate([jnp.sum(a, 0), jnp.sum(g, 0)]).reshape(1, 3 * F)
        s2 = jnp.concatenate([jnp.sum(a * a, 0),
                              jnp.sum(g * g, 0)]).reshape(1, 3 * F)

        @pl.when(i == 0)
        def _():
            ssum_ref[...] = jnp.zeros_like(ssum_ref)
            ssq_ref[...] = jnp.zeros_like(ssq_ref)

        ssum_ref[...] += s1
        ssq_ref[...] += s2

    @pl.when(p == 1)
    def _():
        mu = ssum_ref[...] * (1.0 / B)
        var = ssq_ref[...] * (1.0 / B) - mu * mu
        scale = gamma_ref[...] / jnp.sqrt(var + 1e-5)
        x = jnp.concatenate([agg_ref[...], gen_sc[pl.ds(i * BM, BM), :]],
                            axis=1)
        xn = (x - mu) * scale + beta_ref[...]
        to_ref[...] = jnp.maximum(
            jnp.dot(xn, w1_ref[...], preferred_element_type=F32), 0.0)


def _tc(self_f, agg, env, wg1, bg1, wg2, bg2, gamma, beta, w1):
    grid = (2, NB2)

    def ph0_blk(p, i):
        return (i * (1 - p) + (NB2 - 1) * p, 0)

    def ph1_blk(p, i):
        return (i * p, 0)

    def both_blk(p, i):
        return (i, 0)

    def fixed(p, i):
        return (0, 0)

    return pl.pallas_call(
        _tc_body,
        grid=grid,
        in_specs=[
            pl.BlockSpec((BM, F), ph0_blk),
            pl.BlockSpec((BM, F), both_blk),
            pl.BlockSpec((BM, F), ph0_blk),
            pl.BlockSpec((2 * F, 2 * F), fixed),
            pl.BlockSpec((1, 2 * F), fixed),
            pl.BlockSpec((2 * F, 2 * F), fixed),
            pl.BlockSpec((1, 2 * F), fixed),
            pl.BlockSpec((1, 3 * F), fixed),
            pl.BlockSpec((1, 3 * F), fixed),
            pl.BlockSpec((3 * F, 3 * F), fixed),
        ],
        out_specs=[
            pl.BlockSpec((BM, 2 * F), ph0_blk),
            pl.BlockSpec((BM, 2 * F), ph0_blk),
            pl.BlockSpec((BM, 2 * F), ph0_blk),
            pl.BlockSpec((BM, 2 * F), ph0_blk),
            pl.BlockSpec((BM, 3 * F), ph1_blk),
        ],
        out_shape=[
            jax.ShapeDtypeStruct((B, 2 * F), F32),
            jax.ShapeDtypeStruct((B, 2 * F), F32),
            jax.ShapeDtypeStruct((B, 2 * F), F32),
            jax.ShapeDtypeStruct((B, 2 * F), F32),
            jax.ShapeDtypeStruct((B, 3 * F), F32),
        ],
        scratch_shapes=[
            pltpu.VMEM((B, 2 * F), F32),
            pltpu.VMEM((1, 3 * F), F32),
            pltpu.VMEM((1, 3 * F), F32),
        ],
    )(self_f, agg, env, wg1, bg1.reshape(1, -1), wg2, bg2.reshape(1, -1),
      gamma.reshape(1, -1), beta.reshape(1, -1), w1)


def kernel(table, nodes, to_neighs_list, Wg1, bg1, Wg2, bg2, W1, gamma1,
           beta1):
    assert table.shape == (100000, F) and nodes.shape == (B,)
    assert to_neighs_list.shape == (B, K)
    nghtw = (to_neighs_list.T.reshape(K, NW, EPB)
             .swapaxes(0, 1).reshape(NW, K, EPB))
    self_f, agg, env = _sc_agg(table, nodes, nghtw)
    raw, gen, eraw, egen, to_feats = _tc(self_f, agg, env, Wg1, bg1, Wg2,
                                         bg2, gamma1, beta1, W1)
    return (agg, to_feats, gen, raw, egen, eraw)


# speedup vs baseline: 1.0548x; 1.0011x over previous
import jax
import jax.numpy as jnp
from jax import lax
from jax.experimental import pallas as pl
from jax.experimental.pallas import tpu as pltpu
from jax.experimental.pallas import tpu_sc as plsc

F32 = jnp.float32


def _lane_perm(x, perm):
    return lax.gather(
        x, perm.reshape(16, 1),
        lax.GatherDimensionNumbers(offset_dims=(),
                                   collapsed_slice_dims=(0,),
                                   start_index_map=(0,)),
        (1,), mode=lax.GatherScatterMode.PROMISE_IN_BOUNDS)


B = 16384
K = 16
F = 128
NW = 32
EPB = B // NW
CH = 32
NCHUNK = EPB // CH
NBUF = 2


def _sc_agg_body(table, nodes, nghtw, self_out, agg_out, env_out,
                 nidf_v, nght_v, acc_v, self_v, oself_v, oagg_v, oenv_v,
                 inv_v, cdiv_v, sem_g, sem_o):
    wid = lax.axis_index("s") * 2 + lax.axis_index("c")
    base = wid * EPB

    pltpu.sync_copy(nodes.at[pl.ds(base, EPB)], nidf_v)
    pltpu.sync_copy(nghtw.at[wid], nght_v)

    def cnt_body(g, _):
        sl = pl.ds(g * 16, 16)
        nidv = nidf_v[sl]
        cacc = jnp.zeros((16,), jnp.int32)
        for k in range(K):
            cacc = cacc + jnp.where(nght_v[k, sl] == nidv, 1, 0)
        env_cnt = jnp.maximum(K - cacc, 1).astype(F32)
        ie = 1.0 / env_cnt
        inv_v[sl] = ie
        cdiv_v[sl] = cacc.astype(F32) * ie
        return 0

    lax.fori_loop(0, EPB // 16, cnt_body, 0)

    def zero_body(e, _):
        z = jnp.zeros((16,), F32)
        for b in range(NBUF):
            for j in range(F // 16):
                acc_v[b, e, pl.ds(j * 16, 16)] = z
        return 0

    lax.fori_loop(0, CH, zero_body, 0)

    def in_descs(gi, b):
        ds = pl.ds(gi * CH, CH)
        descs = [pltpu.make_async_copy(
            table.at[nght_v.at[k, ds]], acc_v.at[b], sem_g)
            for k in range(K)]
        descs.append(pltpu.make_async_copy(
            table.at[nidf_v.at[ds]], self_v.at[b], sem_g))
        return descs

    def fire_in(gi, b):
        descs = in_descs(gi, b)
        for d in descs[:K]:
            d.start(add=True)
        descs[K].start()

    def out_descs(gi, b):
        dst = pl.ds(base + gi * CH, CH)
        return (
            pltpu.make_async_copy(oself_v.at[b], self_out.at[dst], sem_o),
            pltpu.make_async_copy(oagg_v.at[b], agg_out.at[dst], sem_o),
            pltpu.make_async_copy(oenv_v.at[b], env_out.at[dst], sem_o),
        )

    def compute(gi, b, ob):
        def ebody(e, _):
            g16 = gi * (CH // 16) + e // 16
            lane = e % 16
            ie = _lane_perm(inv_v[pl.ds(g16 * 16, 16)],
                            jnp.full((16,), lane, jnp.int32))
            ce = _lane_perm(cdiv_v[pl.ds(g16 * 16, 16)],
                            jnp.full((16,), lane, jnp.int32))
            z = jnp.zeros((16,), F32)
            for j in range(F // 16):
                sl = pl.ds(j * 16, 16)
                s = acc_v[b, e, sl]
                selfj = self_v[b, e, sl]
                oself_v[ob, e, sl] = selfj
                oagg_v[ob, e, sl] = s * (1.0 / K)
                oenv_v[ob, e, sl] = s * ie - selfj * ce
                acc_v[b, e, sl] = z
            return 0

        lax.fori_loop(0, CH, ebody, 0)

    for g in range(NBUF - 1):
        fire_in(g, g)

    def ring_body(p, _):
        for bb in range(NBUF):
            gi = p * NBUF + bb
            ob = bb % 2

            @pl.when(gi >= 2)
            def _():
                for d in out_descs(gi - 2, ob):
                    d.wait()

            @pl.when(gi + NBUF - 1 < NCHUNK)
            def _():
                fire_in(gi + NBUF - 1, (bb + NBUF - 1) % NBUF)

            for d in in_descs(gi, bb):
                d.wait()
            compute(gi, bb, ob)
            for d in out_descs(gi, ob):
                d.start()
        return 0

    lax.fori_loop(0, NCHUNK // NBUF, ring_body, 0)
    for gi, ob in ((NCHUNK - 2, 0), (NCHUNK - 1, 1)):
        for d in out_descs(gi, ob):
            d.wait()


def _sc_agg(table, nodes, nghtw):
    mesh = plsc.VectorSubcoreMesh(core_axis_name="c", subcore_axis_name="s")
    out_t = [jax.ShapeDtypeStruct((B, F), F32)] * 3
    scratch = [
        pltpu.VMEM((EPB,), jnp.int32),
        pltpu.VMEM((K, EPB), jnp.int32),
        pltpu.VMEM((NBUF, CH, F), F32),
        pltpu.VMEM((NBUF, CH, F), F32),
        pltpu.VMEM((2, CH, F), F32),
        pltpu.VMEM((2, CH, F), F32),
        pltpu.VMEM((2, CH, F), F32),
        pltpu.VMEM((EPB,), F32),
        pltpu.VMEM((EPB,), F32),
        pltpu.SemaphoreType.DMA,
        pltpu.SemaphoreType.DMA,
    ]
    fn = pl.kernel(_sc_agg_body, out_type=out_t, mesh=mesh,
                   scratch_types=scratch)
    return fn(table, nodes, nghtw)


BM = 2048
NB2 = B // BM


def _tc_body(self_ref, agg_ref, env_ref, wg1_ref, bg1_ref, wg2_ref, bg2_ref,
             gamma_ref, beta_ref, w1_ref,
             raw_ref, gen_ref, eraw_ref, egen_ref, to_ref,
             gen_sc, ssum_ref, ssq_ref):
    p = pl.program_id(0)
    i = pl.program_id(1)

    @pl.when(p == 0)
    def _():
        cat = jnp.concatenate([self_ref[...], agg_ref[...]], axis=1)
        h = jnp.maximum(
            jnp.dot(cat, wg1_ref[...], preferred_element_type=F32)
            + bg1_ref[...], 0.0)
        raw_ref[...] = h
        g = (jnp.dot(h, wg2_ref[...], preferred_element_type=F32)
             + bg2_ref[...])
        gen_ref[...] = g
        gen_sc[pl.ds(i * BM, BM), :] = g
        eh = jnp.maximum(
            jnp.dot(env_ref[...], wg1_ref[F:, :], preferred_element_type=F32)
            + bg1_ref[...], 0.0)
        eraw_ref[...] = eh
        egen_ref[...] = (jnp.dot(eh, wg2_ref[...],
                                 preferred_element_type=F32) + bg2_ref[...])

        a = agg_ref[...]
        s1 = jnp.concatenate([jnp.sum(a, 0), jnp.sum(g, 0)]).reshape(1, 3 * F)
        s2 = jnp.concatenate([jnp.sum(a * a, 0),
                              jnp.sum(g * g, 0)]).reshape(1, 3 * F)

        @pl.when(i == 0)
        def _():
            ssum_ref[...] = jnp.zeros_like(ssum_ref)
            ssq_ref[...] = jnp.zeros_like(ssq_ref)

        ssum_ref[...] += s1
        ssq_ref[...] += s2

    @pl.when(p == 1)
    def _():
        mu = ssum_ref[...] * (1.0 / B)
        var = ssq_ref[...] * (1.0 / B) - mu * mu
        scale = gamma_ref[...] / jnp.sqrt(var + 1e-5)
        x = jnp.concatenate([agg_ref[...], gen_sc[pl.ds(i * BM, BM), :]],
                            axis=1)
        xn = (x - mu) * scale + beta_ref[...]
        to_ref[...] = jnp.maximum(
            jnp.dot(xn, w1_ref[...], preferred_element_type=F32), 0.0)


def _tc(self_f, agg, env, wg1, bg1, wg2, bg2, gamma, beta, w1):
    grid = (2, NB2)

    def ph0_blk(p, i):
        return (i * (1 - p) + (NB2 - 1) * p, 0)

    def ph1_blk(p, i):
        return (i * p, 0)

    def both_blk(p, i):
        return (i, 0)

    def fixed(p, i):
        return (0, 0)

    return pl.pallas_call(
        _tc_body,
        grid=grid,
        in_specs=[
            pl.BlockSpec((BM, F), ph0_blk),
            pl.BlockSpec((BM, F), both_blk),
            pl.BlockSpec((BM, F), ph0_blk),
            pl.BlockSpec((2 * F, 2 * F), fixed),
            pl.BlockSpec((1, 2 * F), fixed),
            pl.BlockSpec((2 * F, 2 * F), fixed),
            pl.BlockSpec((1, 2 * F), fixed),
            pl.BlockSpec((1, 3 * F), fixed),
            pl.BlockSpec((1, 3 * F), fixed),
            pl.BlockSpec((3 * F, 3 * F), fixed),
        ],
        out_specs=[
            pl.BlockSpec((BM, 2 * F), ph0_blk),
            pl.BlockSpec((BM, 2 * F), ph0_blk),
            pl.BlockSpec((BM, 2 * F), ph0_blk),
            pl.BlockSpec((BM, 2 * F), ph0_blk),
            pl.BlockSpec((BM, 3 * F), ph1_blk),
        ],
        out_shape=[
            jax.ShapeDtypeStruct((B, 2 * F), F32),
            jax.ShapeDtypeStruct((B, 2 * F), F32),
            jax.ShapeDtypeStruct((B, 2 * F), F32),
            jax.ShapeDtypeStruct((B, 2 * F), F32),
            jax.ShapeDtypeStruct((B, 3 * F), F32),
        ],
        scratch_shapes=[
            pltpu.VMEM((B, 2 * F), F32),
            pltpu.VMEM((1, 3 * F), F32),
            pltpu.VMEM((1, 3 * F), F32),
        ],
    )(self_f, agg, env, wg1, bg1.reshape(1, -1), wg2, bg2.reshape(1, -1),
      gamma.reshape(1, -1), beta.reshape(1, -1), w1)


def kernel(table, nodes, to_neighs_list, Wg1, bg1, Wg2, bg2, W1, gamma1,
           beta1):
    assert table.shape == (100000, F) and nodes.shape == (B,)
    assert to_neighs_list.shape == (B, K)
    nghtw = (to_neighs_list.T.reshape(K, NW, EPB)
             .swapaxes(0, 1).reshape(NW, K, EPB))
    self_f, agg, env = _sc_agg(table, nodes, nghtw)
    raw, gen, eraw, egen, to_feats = _tc(self_f, agg, env, Wg1, bg1, Wg2,
                                         bg2, gamma1, beta1, W1)
    return (agg, to_feats, gen, raw, egen, eraw)
